# baseline (speedup 1.0000x reference)
import functools

import jax
import jax.numpy as jnp
from jax import lax
from jax.experimental import pallas as pl
from jax.experimental.pallas import tpu as pltpu
from jax.experimental.pallas import tpu_sc as plsc

NG = 50
CUTOFF = 10.0
DIM = 128
ORDER = 5
_DIMS = [DIM // 2 ** i for i in range(ORDER)][::-1]

_NCORES = 2
_NSUB = 16
_NW = _NCORES * _NSUB
_LN2 = 0.6931471805599453
_BN = 2048
_BE = 2048
_EC = 2048

_SC_PARAMS = pltpu.CompilerParams(
    needs_layout_passes=False, use_tc_tiling_on_sc=False)


def _ssp(v):
    return jnp.maximum(v, 0.0) + jnp.log1p(jnp.exp(-jnp.abs(v))) - _LN2


def _pad32(c):
    return max(32, ((c + 31) // 32) * 32)


def _bs_full(shape):
    nd = len(shape)
    return pl.BlockSpec(shape, lambda i, _nd=nd: (0,) * _nd)


def _bs_cols(r, bn):
    return pl.BlockSpec((r, bn), lambda i: (0, i))


def _make_d2(n_nodes, e_pad):
    e_per = e_pad // _NW
    k2 = 2048
    nblk = e_per // k2
    krow = k2 // 128
    mesh = plsc.VectorSubcoreMesh(core_axis_name="c", subcore_axis_name="s")

    @functools.partial(
        pl.kernel,
        out_type=jax.ShapeDtypeStruct((e_pad // 128, 128), jnp.float32),
        mesh=mesh,
        scratch_types=[
            pltpu.VMEM((n_nodes,), jnp.float32),
            pltpu.VMEM((n_nodes,), jnp.float32),
            pltpu.VMEM((n_nodes,), jnp.float32),
            pltpu.VMEM((krow, 128), jnp.int32),
            pltpu.VMEM((krow, 128), jnp.int32),
            pltpu.VMEM((krow, 128), jnp.float32),
        ],
        compiler_params=_SC_PARAMS,
    )
    def d2k(px_hbm, py_hbm, pz_hbm, si_hbm, di_hbm, out_hbm,
            px, py, pz, sidx, didx, d2b):
        c = lax.axis_index("c")
        s = lax.axis_index("s")
        wkr = s * _NCORES + c
        pltpu.sync_copy(px_hbm, px)
        pltpu.sync_copy(py_hbm, py)
        pltpu.sync_copy(pz_hbm, pz)
        base_rows0 = wkr * (e_per // 128)

        def blk(b, carry):
            brow = base_rows0 + b * krow
            pltpu.sync_copy(si_hbm.at[pl.ds(brow, krow)], sidx)
            pltpu.sync_copy(di_hbm.at[pl.ds(brow, krow)], didx)

            def grp(j, carry2):
                for t in range(8):
                    sl = pl.ds(t * 16, 16)
                    vs = sidx[j, sl]
                    vd = didx[j, sl]
                    dx = plsc.load_gather(px, [vs]) - plsc.load_gather(px, [vd])
                    dy = plsc.load_gather(py, [vs]) - plsc.load_gather(py, [vd])
                    dz = plsc.load_gather(pz, [vs]) - plsc.load_gather(pz, [vd])
                    d2b[j, sl] = dx * dx + dy * dy + dz * dz
                return carry2

            lax.fori_loop(0, krow, grp, 0)
            pltpu.sync_copy(d2b, out_hbm.at[pl.ds(brow, krow)])
            return carry

        lax.fori_loop(0, nblk, blk, 0)

    return d2k


def _make_gms(n_nodes, n_pad, e_pad, wsc):
    cpt = wsc // _NW
    ec = {1: 16384, 2: 8192, 4: 4096}[cpt]
    k = ec // 2048
    nch = e_pad // ec
    mesh = plsc.VectorSubcoreMesh(core_axis_name="c", subcore_axis_name="s")

    @functools.partial(
        pl.kernel,
        out_type=jax.ShapeDtypeStruct((wsc, n_pad), jnp.float32),
        mesh=mesh,
        scratch_types=[
            pltpu.VMEM((cpt, n_nodes), jnp.float32),
            pltpu.VMEM((cpt, n_pad), jnp.float32),
            pltpu.VMEM((2, k * 32, 128), jnp.int32),
            pltpu.VMEM((2, cpt, ec), jnp.float32),
            pltpu.SemaphoreType.DMA,
            pltpu.SemaphoreType.DMA,
        ],
        compiler_params=_SC_PARAMS,
    )
    def gms(ht_hbm, wt_hbm, sdi_hbm, out_hbm,
            hcol, acol, ibuf, wbuf, sm0, sm1):
        c = lax.axis_index("c")
        s = lax.axis_index("s")
        col0 = (s * _NCORES + c) * cpt
        sems = (sm0, sm1)
        spl = [jnp.full((16,), q, jnp.int32) for q in range(cpt)]

        pltpu.sync_copy(ht_hbm.at[pl.ds(col0, cpt), pl.ds(0, n_nodes)], hcol)

        def zcol(i, carry):
            for q in range(cpt):
                acol[q, pl.ds(i * 16, 16)] = jnp.zeros((16,), jnp.float32)
            return carry

        lax.fori_loop(0, n_pad // 16, zcol, 0)

        def fire(ch, buf):
            pltpu.async_copy(sdi_hbm.at[pl.ds(ch * k * 32, k * 32)],
                             ibuf.at[buf], sems[buf])
            pltpu.async_copy(wt_hbm.at[pl.ds(col0, cpt), pl.ds(ch * ec, ec)],
                             wbuf.at[buf], sems[buf])

        def consume(ch, buf):
            pltpu.make_async_copy(sdi_hbm.at[pl.ds(ch * k * 32, k * 32)],
                                  ibuf.at[buf], sems[buf]).wait()
            pltpu.make_async_copy(
                wt_hbm.at[pl.ds(col0, cpt), pl.ds(ch * ec, ec)],
                wbuf.at[buf], sems[buf]).wait()

            def rowloop(rr, carry):
                for j in range(k):
                    for gg in range(8):
                        sl = pl.ds(gg * 16, 16)
                        vs = ibuf[buf, j * 32 + rr, sl]
                        vd = ibuf[buf, j * 32 + 16 + rr, sl]
                        woff = j * 2048 + rr * 128 + gg * 16
                        for q in range(cpt):
                            hv = plsc.load_gather(hcol, [spl[q], vs])
                            wv = wbuf[buf, q, pl.ds(woff, 16)]
                            plsc.addupdate_scatter(acol, [spl[q], vd], hv * wv)
                return carry

            lax.fori_loop(0, 16, rowloop, 0)

        fire(0, 0)

        def pair(g, carry):
            ch = 2 * g
            fire(ch + 1, 1)
            consume(ch, 0)
            fire(ch + 2, 0)
            consume(ch + 1, 1)
            return carry

        lax.fori_loop(0, nch // 2 - 1, pair, 0)
        fire(nch - 1, 1)
        consume(nch - 2, 0)
        consume(nch - 1, 1)

        pltpu.sync_copy(acol, out_hbm.at[pl.ds(col0, cpt)])

    return gms


def _filter_w(d2r, wn1t, bn1c, wn2t, bn2c, n_edges, wsc, ncalls):
    e_pad = d2r.shape[1]
    cp = wn1t.shape[0]
    delta = CUTOFF / (NG - 1)
    coeff = -0.5 / delta ** 2

    def body(d2_ref, wn1_ref, bn1_ref, wn2_ref, bn2_ref, *outs):
        i = pl.program_id(0)
        d = jnp.sqrt(d2_ref[...] + 1e-12)
        offs = delta * lax.broadcasted_iota(jnp.int32, (NG, 1), 0).astype(
            jnp.float32)
        diff = d - offs
        rbft = jnp.exp(coeff * (diff * diff))
        ut = _ssp(jnp.dot(wn1_ref[...], rbft,
                          preferred_element_type=jnp.float32) + bn1_ref[...])
        wt = _ssp(jnp.dot(wn2_ref[...], ut,
                          preferred_element_type=jnp.float32) + bn2_ref[...])
        cols = i * _BE + lax.broadcasted_iota(jnp.int32, (1, _BE), 1)
        wt = jnp.where(cols < n_edges, wt, 0.0)
        for t in range(ncalls):
            outs[t][...] = wt[t * wsc:(t + 1) * wsc]

    return pl.pallas_call(
        body,
        grid=(e_pad // _BE,),
        in_specs=[
            pl.BlockSpec((1, _BE), lambda i: (0, i)),
            _bs_full((cp, NG)),
            _bs_full((cp, 1)),
            _bs_full((cp, cp)),
            _bs_full((cp, 1)),
        ],
        out_specs=[_bs_cols(wsc, _BE) for _ in range(ncalls)],
        out_shape=[jax.ShapeDtypeStruct((wsc, e_pad), jnp.float32)
                   for _ in range(ncalls)],
    )(d2r, wn1t, bn1c, wn2t, bn2c)


def _t0(x3dt, w1t, wsc, ncalls):
    n = x3dt.shape[1]
    cin = x3dt.shape[0]

    def body(x_ref, w_ref, *outs):
        ht = jnp.dot(w_ref[...], x_ref[...], preferred_element_type=jnp.float32)
        for t in range(ncalls):
            outs[t][...] = ht[t * wsc:(t + 1) * wsc]

    return pl.pallas_call(
        body,
        grid=(n // _BN,),
        in_specs=[_bs_cols(cin, _BN), _bs_full(w1t.shape)],
        out_specs=[_bs_cols(wsc, _BN) for _ in range(ncalls)],
        out_shape=[jax.ShapeDtypeStruct((wsc, n), jnp.float32)
                   for _ in range(ncalls)],
    )(x3dt, w1t)


def _transition(aggs, wsc, w2cht, b2c, w1nt, wscn, ncn, n, *, pwa=None,
                dw=None, lohi=None, emit_pwa=False, emit_dw=False,
                final=False):
    na = len(aggs)
    sd = sum(_DIMS)

    def body(*refs):
        refs = list(refs)
        ar = refs[:na]
        w2r = refs[na:2 * na]
        pos = 2 * na
        b2r = refs[pos]
        pos += 1
        pwar = dwr = w1r = None
        if emit_dw:
            pwar = refs[pos]
            pos += 1
        if lohi is not None:
            dwr = refs[pos]
            pos += 1
        if not final:
            w1r = refs[pos]
            pos += 1
        outs = refs[pos:]
        acc = b2r[...]
        for t in range(na):
            acc = acc + jnp.dot(w2r[t][...], ar[t][...],
                                preferred_element_type=jnp.float32)
        o = _ssp(acc)
        if final:
            outs[0][...] = o
            return
        if emit_pwa:
            outs[ncn][...] = o[: _DIMS[0]]
            hin = o[_DIMS[0]:]
        elif emit_dw:
            outs[ncn][...] = o
            hin = pwar[...] * o[: _DIMS[0]]
        else:
            lo, hi = lohi
            hin = o * dwr[...][lo:hi]
        hnt = jnp.dot(w1r[...], hin, preferred_element_type=jnp.float32)
        for t in range(ncn):
            outs[t][...] = hnt[t * wscn:(t + 1) * wscn]

    in_specs = []
    args = []
    for a in aggs:
        in_specs.append(_bs_cols(wsc, _BN))
        args.append(a)
    for wc in w2cht:
        in_specs.append(_bs_full(wc.shape))
        args.append(wc)
    in_specs.append(_bs_full(b2c.shape))
    args.append(b2c)
    if emit_dw:
        in_specs.append(_bs_cols(_DIMS[0], _BN))
        args.append(pwa)
    if lohi is not None:
        in_specs.append(_bs_cols(sd, _BN))
        args.append(dw)
    if not final:
        in_specs.append(_bs_full(w1nt.shape))
        args.append(w1nt)

    out_specs = []
    out_shape = []
    if final:
        cout = w2cht[0].shape[0]
        out_specs.append(_bs_cols(cout, _BN))
        out_shape.append(jax.ShapeDtypeStruct((cout, n), jnp.float32))
    else:
        for _ in range(ncn):
            out_specs.append(_bs_cols(wscn, _BN))
            out_shape.append(jax.ShapeDtypeStruct((wscn, n), jnp.float32))
        if emit_pwa:
            out_specs.append(_bs_cols(_DIMS[0], _BN))
            out_shape.append(jax.ShapeDtypeStruct((_DIMS[0], n), jnp.float32))
        elif emit_dw:
            out_specs.append(_bs_cols(sd, _BN))
            out_shape.append(jax.ShapeDtypeStruct((sd, n), jnp.float32))

    res = pl.pallas_call(
        body,
        grid=(n // _BN,),
        in_specs=in_specs,
        out_specs=out_specs,
        out_shape=out_shape,
    )(*args)
    if final:
        return res[0]
    return list(res)


def _prep_filter(lp, cout):
    cp = _pad32(cout)
    wn1t = jnp.pad(lp["Wn1"].T, ((0, cp - cout), (0, 0)))
    bn1c = jnp.pad(lp["bn1"], (0, cp - cout)).reshape(cp, 1)
    wn2t = jnp.pad(lp["Wn2"].T, ((0, cp - cout), (0, cp - cout)))
    bn2c = jnp.pad(lp["bn2"], (0, cp - cout)).reshape(cp, 1)
    return wn1t, bn1c, wn2t, bn2c


def _prep_out(lp, cout, wsc, ncalls):
    cp = _pad32(cout)
    w2pt = jnp.pad(lp["W2"], ((0, cp - cout), (0, 0))).T
    chunks = [w2pt[:, t * wsc:(t + 1) * wsc] for t in range(ncalls)]
    return chunks, lp["b2"].reshape(cout, 1)


def kernel(x, x_3d, pos, edge_index, params):
    n = x_3d.shape[0]
    e = edge_index.shape[1]
    e_pad = ((e + 32767) // 32768) * 32768
    n_pad = ((n + 2047) // 2048) * 2048

    ei = jnp.pad(edge_index, ((0, 0), (0, e_pad - e)))
    si = ei[0].reshape(e_pad // 128, 128)
    di = ei[1].reshape(e_pad // 128, 128)
    d2 = _make_d2(n, e_pad)(pos[:, 0], pos[:, 1], pos[:, 2], si, di)
    d2r = d2.reshape(1, e_pad)
    sdi = ei.reshape(2, e_pad // 2048, 16, 128).transpose(1, 0, 2, 3)
    sdi = sdi.reshape(e_pad // 2048 * 32, 128)

    names = ["proj_in", "dwconv", "pw0", "pw1", "pw2", "pw3", "proj_out"]
    couts = [2 * DIM, sum(_DIMS)] + [_DIMS[i + 1] for i in range(ORDER - 1)] + [DIM]
    cps = [_pad32(c) for c in couts]
    wscs = [min(128, cp) for cp in cps]
    ncs = [cp // w for cp, w in zip(cps, wscs)]
    gms = {}
    for w in set(wscs):
        gms[w] = _make_gms(n, n_pad, e_pad, w)

    wsp = []
    for li in range(7):
        fp = _prep_filter(params[names[li]], couts[li])
        wsp.append(_filter_w(d2r, *fp, e, wscs[li], ncs[li]))

    def run_edge(li, h2s):
        return [gms[wscs[li]](h2s[t], wsp[li][t], sdi)
                for t in range(ncs[li])]

    bounds = []
    start = 0
    for dcur in _DIMS:
        bounds.append((start, start + dcur))
        start += dcur

    p1 = params["proj_in"]
    x3dt = jnp.pad(x_3d.T, ((0, 0), (0, n_pad - n)))
    h2s = _t0(x3dt, p1["W1"].T, wscs[0], ncs[0])
    aggs = run_edge(0, h2s)

    w2ch, b2c = _prep_out(p1, couts[0], wscs[0], ncs[0])
    pdw = params["dwconv"]
    w1dwt = jnp.pad(pdw["W1"].T, ((0, cps[1] - couts[1]), (0, 0)))
    *h2s, pwa = _transition(aggs, wscs[0], w2ch, b2c, w1dwt, wscs[1], ncs[1],
                            n_pad, emit_pwa=True)
    aggs = run_edge(1, h2s)

    w2ch, b2c = _prep_out(pdw, couts[1], wscs[1], ncs[1])
    w1nt = jnp.pad(params["pw0"]["W1"].T, ((0, cps[2] - couts[2]), (0, 0)))
    *h2s, dw = _transition(aggs, wscs[1], w2ch, b2c, w1nt, wscs[2], ncs[2],
                           n_pad, pwa=pwa, emit_dw=True)
    aggs = run_edge(2, h2s)

    for i in range(3, 7):
        w2ch, b2c = _prep_out(params[names[i - 1]], couts[i - 1],
                              wscs[i - 1], ncs[i - 1])
        w1nt = params[names[i]]["W1"].T
        if cps[i] != couts[i]:
            w1nt = jnp.pad(w1nt, ((0, cps[i] - couts[i]), (0, 0)))
        h2s = _transition(aggs, wscs[i - 1], w2ch, b2c, w1nt, wscs[i], ncs[i],
                          n_pad, dw=dw, lohi=bounds[i - 2])
        aggs = run_edge(i, h2s)

    w2ch, b2c = _prep_out(params["proj_out"], couts[6], wscs[6], ncs[6])
    out_t = _transition(aggs, wscs[6], w2ch, b2c, None, 0, 0, n_pad, final=True)
    return out_t[:, :n].T


# speedup vs baseline: 1.0769x; 1.0769x over previous
import functools

import jax
import jax.numpy as jnp
from jax import lax
from jax.experimental import pallas as pl
from jax.experimental.pallas import tpu as pltpu
from jax.experimental.pallas import tpu_sc as plsc

NG = 50
CUTOFF = 10.0
DIM = 128
ORDER = 5
_DIMS = [DIM // 2 ** i for i in range(ORDER)][::-1]

_NCORES = 2
_NSUB = 16
_LN2 = 0.6931471805599453
_BN = 2000

_SC_PARAMS = pltpu.CompilerParams(
    needs_layout_passes=False, use_tc_tiling_on_sc=False)


def _ssp(v):
    return jnp.maximum(v, 0.0) + jnp.log1p(jnp.exp(-jnp.abs(v))) - _LN2


def _pad32(c):
    return max(32, ((c + 31) // 32) * 32)


def _bs_rows(bn, c):
    return pl.BlockSpec((bn, c), lambda i: (i, 0))


def _bs_full(shape):
    nd = len(shape)
    return pl.BlockSpec(shape, lambda i, _nd=nd: (0,) * _nd)


def _make_d2(n_nodes, e_pad):
    e_per = e_pad // (_NCORES * _NSUB)
    k2 = 2048
    nblk = e_per // k2
    krow = k2 // 128
    mesh = plsc.VectorSubcoreMesh(core_axis_name="c", subcore_axis_name="s")

    @functools.partial(
        pl.kernel,
        out_type=jax.ShapeDtypeStruct((e_pad // 128, 128), jnp.float32),
        mesh=mesh,
        scratch_types=[
            pltpu.VMEM((n_nodes,), jnp.float32),
            pltpu.VMEM((n_nodes,), jnp.float32),
            pltpu.VMEM((n_nodes,), jnp.float32),
            pltpu.VMEM((krow, 128), jnp.int32),
            pltpu.VMEM((krow, 128), jnp.int32),
            pltpu.VMEM((krow, 128), jnp.float32),
        ],
        compiler_params=_SC_PARAMS,
    )
    def d2k(px_hbm, py_hbm, pz_hbm, si_hbm, di_hbm, out_hbm,
            px, py, pz, sidx, didx, d2b):
        c = lax.axis_index("c")
        s = lax.axis_index("s")
        wkr = s * _NCORES + c
        pltpu.sync_copy(px_hbm, px)
        pltpu.sync_copy(py_hbm, py)
        pltpu.sync_copy(pz_hbm, pz)
        base_rows0 = wkr * (e_per // 128)

        def blk(b, carry):
            brow = base_rows0 + b * krow
            pltpu.sync_copy(si_hbm.at[pl.ds(brow, krow)], sidx)
            pltpu.sync_copy(di_hbm.at[pl.ds(brow, krow)], didx)

            def grp(j, carry2):
                for t in range(8):
                    sl = pl.ds(t * 16, 16)
                    vs = sidx[j, sl]
                    vd = didx[j, sl]
                    dx = plsc.load_gather(px, [vs]) - plsc.load_gather(px, [vd])
                    dy = plsc.load_gather(py, [vs]) - plsc.load_gather(py, [vd])
                    dz = plsc.load_gather(pz, [vs]) - plsc.load_gather(pz, [vd])
                    d2b[j, sl] = dx * dx + dy * dy + dz * dz
                return carry2

            lax.fori_loop(0, krow, grp, 0)
            pltpu.sync_copy(d2b, out_hbm.at[pl.ds(brow, krow)])
            return carry

        lax.fori_loop(0, nblk, blk, 0)

    return d2k


_GMS_K = {128: 128, 64: 512, 32: 1024, 16: 2048}
_G = 2048


def _make_gms(n_nodes, n_pad, e_pad, hc):
    k = _GMS_K[hc]
    nsb = _G // k
    kb = k // 128
    e_per = e_pad // _NSUB
    nblk = e_per // _G
    g_rows = _G // 128
    rows_per = n_pad // _NSUB
    zr = 32
    nz = rows_per // zr
    mesh = plsc.VectorSubcoreMesh(core_axis_name="c", subcore_axis_name="s")

    @functools.partial(
        pl.kernel,
        out_type=jax.ShapeDtypeStruct((2, n_pad, hc), jnp.float32),
        mesh=mesh,
        scratch_types=[
            pltpu.VMEM((g_rows, 128), jnp.int32),
            pltpu.VMEM((g_rows, 128), jnp.int32),
            pltpu.VMEM((k, hc), jnp.float32),
            pltpu.VMEM((k, hc), jnp.float32),
            pltpu.VMEM((zr, hc), jnp.float32),
            pltpu.VMEM_SHARED((n_pad, hc), jnp.float32),
            pltpu.SemaphoreType.DMA,
        ],
        compiler_params=_SC_PARAMS,
    )
    def gms(h2_hbm, w2_hbm, si_hbm, di_hbm, out_hbm,
            sidx, didx, rows, wrows, zbuf, agg, sem):
        c = lax.axis_index("c")
        s = lax.axis_index("s")
        coff = c * n_nodes

        def zrow(i, carry):
            for j in range(hc // 16):
                zbuf[i, pl.ds(j * 16, 16)] = jnp.zeros((16,), jnp.float32)
            return carry

        lax.fori_loop(0, zr, zrow, 0)
        r0 = s * rows_per
        for t in range(nz):
            pltpu.sync_copy(zbuf, agg.at[pl.ds(r0 + t * zr, zr)])
        plsc.subcore_barrier()

        base_rows0 = s * (e_per // 128)

        def eblk(b, carry):
            brow = base_rows0 + b * g_rows
            pltpu.sync_copy(si_hbm.at[pl.ds(brow, g_rows)], sidx)
            pltpu.sync_copy(di_hbm.at[pl.ds(brow, g_rows)], didx)

            def shift(j, carry2):
                for t in range(8):
                    sl = pl.ds(t * 16, 16)
                    sidx[j, sl] = sidx[j, sl] + coff
                return carry2

            lax.fori_loop(0, g_rows, shift, 0)
            for sb in range(nsb):
                cps = [
                    pltpu.async_copy(h2_hbm.at[sidx.at[sb * kb + j]],
                                     rows.at[pl.ds(j * 128, 128)], sem)
                    for j in range(kb)
                ]
                pltpu.sync_copy(
                    w2_hbm.at[pl.ds(c * e_pad + s * e_per + b * _G + sb * k,
                                    k)],
                    wrows)
                for cp_ in cps:
                    cp_.wait()
                urr = max(1, 16 // (hc // 16))

                def mulrow(g, carry2):
                    for rr in range(urr):
                        r = g * urr + rr
                        for j in range(hc // 16):
                            sl = pl.ds(j * 16, 16)
                            rows[r, sl] = rows[r, sl] * wrows[r, sl]
                    return carry2

                lax.fori_loop(0, k // urr, mulrow, 0)
                for j in range(kb):
                    pltpu.sync_copy(rows.at[pl.ds(j * 128, 128)],
                                    agg.at[didx.at[sb * kb + j]], add=True)
            return carry

        lax.fori_loop(0, nblk, eblk, 0)
        plsc.subcore_barrier()
        pltpu.sync_copy(agg.at[pl.ds(r0, rows_per)],
                        out_hbm.at[c, pl.ds(r0, rows_per)])

    return gms


def _filter_w(d2c, wn1, bn1, wn2, bn2, n_edges, hc, be=2048):
    e_pad = d2c.shape[0]
    cp = wn1.shape[1]
    delta = CUTOFF / (NG - 1)
    coeff = -0.5 / delta ** 2

    def body(d2_ref, wn1_ref, bn1_ref, wn2_ref, bn2_ref, out_ref):
        i = pl.program_id(0)
        d = jnp.sqrt(d2_ref[...] + 1e-12)
        offs = delta * lax.broadcasted_iota(jnp.int32, (1, NG), 1).astype(
            jnp.float32)
        diff = d - offs
        rbf = jnp.exp(coeff * (diff * diff))
        u = _ssp(jnp.dot(rbf, wn1_ref[...],
                         preferred_element_type=jnp.float32) + bn1_ref[...])
        w = _ssp(jnp.dot(u, wn2_ref[...],
                         preferred_element_type=jnp.float32) + bn2_ref[...])
        rows = i * be + lax.broadcasted_iota(jnp.int32, (be, 1), 0)
        w = jnp.where(rows < n_edges, w, 0.0)
        out_ref[0] = w[:, :hc]
        out_ref[1] = w[:, hc:]

    return pl.pallas_call(
        body,
        grid=(e_pad // be,),
        in_specs=[
            pl.BlockSpec((be, 1), lambda i: (i, 0)),
            _bs_full((NG, cp)),
            _bs_full((1, cp)),
            _bs_full((cp, cp)),
            _bs_full((1, cp)),
        ],
        out_specs=pl.BlockSpec((2, be, hc), lambda i: (0, i, 0)),
        out_shape=jax.ShapeDtypeStruct((2, e_pad, hc), jnp.float32),
    )(d2c, wn1, bn1, wn2, bn2)


def _t0(x3d, w1p, hc):
    n = x3d.shape[0]
    cin = x3d.shape[1]

    def body(x_ref, w_ref, out_ref):
        h = jnp.dot(x_ref[...], w_ref[...], preferred_element_type=jnp.float32)
        out_ref[0] = h[:, :hc]
        out_ref[1] = h[:, hc:]

    return pl.pallas_call(
        body,
        grid=(n // _BN,),
        in_specs=[_bs_rows(_BN, cin), _bs_full(w1p.shape)],
        out_specs=pl.BlockSpec((2, _BN, hc), lambda i: (0, i, 0)),
        out_shape=jax.ShapeDtypeStruct((2, n, hc), jnp.float32),
    )(x3d, w1p)


def _post(hc):
    return [
        pl.BlockSpec((1, _BN, hc), lambda i: (0, i, 0)),
        pl.BlockSpec((1, _BN, hc), lambda i: (1, i, 0)),
    ]


def _t1(agg, w2a, w2b, b2, w1n, hc, hcn, n):

    def body(aa, ab, wa, wb, b2r, w1r, hout, pout):
        o = _ssp(jnp.dot(aa[0], wa[...], preferred_element_type=jnp.float32)
                 + jnp.dot(ab[0], wb[...], preferred_element_type=jnp.float32)
                 + b2r[...])
        pout[...] = o[:, : _DIMS[0]]
        hn = jnp.dot(o[:, _DIMS[0]:], w1r[...],
                     preferred_element_type=jnp.float32)
        hout[0] = hn[:, :hcn]
        hout[1] = hn[:, hcn:]

    return pl.pallas_call(
        body,
        grid=(n // _BN,),
        in_specs=_post(hc) + [
            _bs_full(w2a.shape), _bs_full(w2b.shape), _bs_full(b2.shape),
            _bs_full(w1n.shape),
        ],
        out_specs=[
            pl.BlockSpec((2, _BN, hcn), lambda i: (0, i, 0)),
            _bs_rows(_BN, _DIMS[0]),
        ],
        out_shape=[
            jax.ShapeDtypeStruct((2, n, hcn), jnp.float32),
            jax.ShapeDtypeStruct((n, _DIMS[0]), jnp.float32),
        ],
    )(agg, agg, w2a, w2b, b2, w1n)


def _t2(agg, w2a, w2b, b2, pwa, w1n, hc, hcn, n):
    sd = sum(_DIMS)

    def body(aa, ab, wa, wb, b2r, pr, w1r, hout, dwout):
        o = _ssp(jnp.dot(aa[0], wa[...], preferred_element_type=jnp.float32)
                 + jnp.dot(ab[0], wb[...], preferred_element_type=jnp.float32)
                 + b2r[...])
        dwout[...] = o
        h = pr[...] * o[:, : _DIMS[0]]
        hn = jnp.dot(h, w1r[...], preferred_element_type=jnp.float32)
        hout[0] = hn[:, :hcn]
        hout[1] = hn[:, hcn:]

    return pl.pallas_call(
        body,
        grid=(n // _BN,),
        in_specs=_post(hc) + [
            _bs_full(w2a.shape), _bs_full(w2b.shape), _bs_full(b2.shape),
            _bs_rows(_BN, _DIMS[0]), _bs_full(w1n.shape),
        ],
        out_specs=[
            pl.BlockSpec((2, _BN, hcn), lambda i: (0, i, 0)),
            _bs_rows(_BN, sd),
        ],
        out_shape=[
            jax.ShapeDtypeStruct((2, n, hcn), jnp.float32),
            jax.ShapeDtypeStruct((n, sd), jnp.float32),
        ],
    )(agg, agg, w2a, w2b, b2, pwa, w1n)


def _tmid(agg, w2a, w2b, b2, dw, lo, hi, w1n, hc, hcn, n):
    sd = sum(_DIMS)

    def body(aa, ab, wa, wb, b2r, dwr, w1r, hout):
        o = _ssp(jnp.dot(aa[0], wa[...], preferred_element_type=jnp.float32)
                 + jnp.dot(ab[0], wb[...], preferred_element_type=jnp.float32)
                 + b2r[...])
        h = o * dwr[...][:, lo:hi]
        hn = jnp.dot(h, w1r[...], preferred_element_type=jnp.float32)
        hout[0] = hn[:, :hcn]
        hout[1] = hn[:, hcn:]

    return pl.pallas_call(
        body,
        grid=(n // _BN,),
        in_specs=_post(hc) + [
            _bs_full(w2a.shape), _bs_full(w2b.shape), _bs_full(b2.shape),
            _bs_rows(_BN, sd), _bs_full(w1n.shape),
        ],
        out_specs=pl.BlockSpec((2, _BN, hcn), lambda i: (0, i, 0)),
        out_shape=jax.ShapeDtypeStruct((2, n, hcn), jnp.float32),
    )(agg, agg, w2a, w2b, b2, dw, w1n)


def _t7(agg, w2a, w2b, b2, hc, n):
    cout = w2a.shape[1]

    def body(aa, ab, wa, wb, b2r, out_ref):
        out_ref[...] = _ssp(
            jnp.dot(aa[0], wa[...], preferred_element_type=jnp.float32)
            + jnp.dot(ab[0], wb[...], preferred_element_type=jnp.float32)
            + b2r[...])

    return pl.pallas_call(
        body,
        grid=(n // _BN,),
        in_specs=_post(hc) + [
            _bs_full(w2a.shape), _bs_full(w2b.shape), _bs_full(b2.shape),
        ],
        out_specs=_bs_rows(_BN, cout),
        out_shape=jax.ShapeDtypeStruct((n, cout), jnp.float32),
    )(agg, agg, w2a, w2b, b2)


def _prep_filter(lp, cout):
    cp = _pad32(cout)
    wn1 = jnp.pad(lp["Wn1"], ((0, 0), (0, cp - cout)))
    bn1 = jnp.pad(lp["bn1"], (0, cp - cout)).reshape(1, cp)
    wn2 = jnp.pad(lp["Wn2"], ((0, cp - cout), (0, cp - cout)))
    bn2 = jnp.pad(lp["bn2"], (0, cp - cout)).reshape(1, cp)
    return wn1, bn1, wn2, bn2


def _prep_out(lp, cout):
    cp = _pad32(cout)
    hc = cp // 2
    w2p = jnp.pad(lp["W2"], ((0, cp - cout), (0, 0)))
    return w2p[:hc], w2p[hc:], lp["b2"].reshape(1, cout)


def kernel(x, x_3d, pos, edge_index, params):
    n = x_3d.shape[0]
    e = edge_index.shape[1]
    e_pad = ((e + 32767) // 32768) * 32768
    n_pad = ((n + 2047) // 2048) * 2048

    ei = jnp.pad(edge_index, ((0, 0), (0, e_pad - e)))
    si = ei[0].reshape(e_pad // 128, 128)
    di = ei[1].reshape(e_pad // 128, 128)
    d2 = _make_d2(n, e_pad)(pos[:, 0], pos[:, 1], pos[:, 2], si, di)
    d2c = d2.reshape(e_pad, 1)

    names = ["proj_in", "dwconv", "pw0", "pw1", "pw2", "pw3", "proj_out"]
    couts = [2 * DIM, sum(_DIMS)] + [_DIMS[i + 1] for i in range(ORDER - 1)] + [DIM]
    cps = [_pad32(c) for c in couts]
    hcs = [cp // 2 for cp in cps]
    gms = {}
    for hc in set(hcs):
        gms[hc] = _make_gms(n, n_pad, e_pad, hc)

    wsp = []
    for li in range(7):
        fp = _prep_filter(params[names[li]], couts[li])
        wsp.append(_filter_w(d2c, *fp, e, hcs[li]))

    def run_edge(li, h2):
        return gms[hcs[li]](h2.reshape(2 * n, hcs[li]),
                            wsp[li].reshape(2 * e_pad, hcs[li]), si, di)

    bounds = []
    start = 0
    for dcur in _DIMS:
        bounds.append((start, start + dcur))
        start += dcur

    p1 = params["proj_in"]
    h2 = _t0(x_3d, p1["W1"], hcs[0])
    agg = run_edge(0, h2)

    w2a, w2b, b2 = _prep_out(p1, couts[0])
    pdw = params["dwconv"]
    w1dw = jnp.pad(pdw["W1"], ((0, 0), (0, cps[1] - couts[1])))
    h2, pwa = _t1(agg, w2a, w2b, b2, w1dw, hcs[0], hcs[1], n)
    agg = run_edge(1, h2)

    w2a, w2b, b2 = _prep_out(pdw, couts[1])
    w1n = jnp.pad(params["pw0"]["W1"], ((0, 0), (0, cps[2] - couts[2])))
    h2, dw = _t2(agg, w2a, w2b, b2, pwa, w1n, hcs[1], hcs[2], n)
    agg = run_edge(2, h2)

    for i in range(3, 7):
        w2a, w2b, b2 = _prep_out(params[names[i - 1]], couts[i - 1])
        w1n = params[names[i]]["W1"]
        if cps[i] != couts[i]:
            w1n = jnp.pad(w1n, ((0, 0), (0, cps[i] - couts[i])))
        lo, hi = bounds[i - 2]
        h2 = _tmid(agg, w2a, w2b, b2, dw, lo, hi, w1n, hcs[i - 1], hcs[i], n)
        agg = run_edge(i, h2)

    w2a, w2b, b2 = _prep_out(params["proj_out"], couts[6])
    return _t7(agg, w2a, w2b, b2, hcs[6], n)
